# baseline (speedup 1.0000x reference)
import functools

import jax
import jax.numpy as jnp
from jax import lax
from jax.experimental import pallas as pl
from jax.experimental.pallas import tpu as pltpu
from jax.experimental.pallas import tpu_sc as plsc

NC = 2
NS = 16
NW = NC * NS


_ROW_BLK = 8192


def _transform_body(pcat_ref, wt_ref, b_ref, out_ref):
    out_ref[...] = (
        lax.dot_general(pcat_ref[...], wt_ref[...],
                        dimension_numbers=(((0,), (0,)), ((), ())),
                        preferred_element_type=jnp.float32,
                        precision=lax.Precision.HIGHEST)
        + b_ref[...]
    )


def _transform_table(pcat, wt, b2d):
    d, v = pcat.shape
    grid = (v + _ROW_BLK - 1) // _ROW_BLK
    return pl.pallas_call(
        _transform_body,
        grid=(grid,),
        in_specs=[
            pl.BlockSpec((d, _ROW_BLK), lambda i: (0, i)),
            pl.BlockSpec((d, d), lambda i: (0, 0)),
            pl.BlockSpec((1, d), lambda i: (0, 0)),
        ],
        out_specs=pl.BlockSpec((_ROW_BLK, d), lambda i: (i, 0)),
        out_shape=jax.ShapeDtypeStruct((v, d), jnp.float32),
    )(pcat, wt, b2d)


_BPW = 128
_CH = 100
_RING = 4


def _make_gather(bsz, seq, d):
    n_chunks = _BPW * seq // _CH
    mesh = plsc.VectorSubcoreMesh(
        core_axis_name="c", subcore_axis_name="s",
        num_cores=NC, num_subcores=NS)

    @functools.partial(
        pl.kernel,
        out_type=jax.ShapeDtypeStruct((bsz, seq, d), jnp.float32),
        mesh=mesh,
        scratch_types=[
            pltpu.VMEM((n_chunks, _CH), jnp.int32),
            pltpu.VMEM((_RING, seq, d), jnp.float32),
        ] + [pltpu.SemaphoreType.DMA] * _RING,
        compiler_params=pltpu.CompilerParams(use_tc_tiling_on_sc=False),
    )
    def gather(table_hbm, idx_hbm, out_hbm, idx_v, ring, *sems):
        wid = lax.axis_index("s") * NC + lax.axis_index("c")
        b0 = wid * _BPW
        pltpu.sync_copy(idx_hbm.at[wid], idx_v)

        def fire(bj, r):
            for h in range(2):
                pltpu.async_copy(
                    table_hbm.at[idx_v.at[bj * 2 + h]],
                    ring.at[r, pl.ds(h * _CH, _CH)],
                    sems[r])

        def drain(r):
            for h in range(2):
                pltpu.make_async_copy(
                    table_hbm.at[idx_v.at[0]],
                    ring.at[r, pl.ds(h * _CH, _CH)],
                    sems[r]).wait()

        def write(bj, r):
            pltpu.sync_copy(ring.at[r], out_hbm.at[b0 + bj])

        for bj in range(_RING - 1):
            fire(bj, bj)

        def body(i2, carry):
            for r in range(_RING):
                bj = i2 * _RING + r
                fire(bj + _RING - 1, (r + _RING - 1) % _RING)
                drain(r)
                write(bj, r)
            return carry

        lax.fori_loop(0, _BPW // _RING - 1, body, 0)
        i2 = _BPW // _RING - 1
        for r in range(_RING):
            bj = i2 * _RING + r
            if bj + _RING - 1 < _BPW:
                fire(bj + _RING - 1, (r + _RING - 1) % _RING)
            drain(r)
            write(bj, r)

    return gather


def kernel(nodes, pca_matrix, W, b):
    bsz, seq = nodes.shape
    d = pca_matrix.shape[1]

    table = _transform_table(pca_matrix.T, W.T, b.reshape(1, d))

    n_chunks = _BPW * seq // _CH
    idx = nodes.astype(jnp.int32).reshape(NW, n_chunks, _CH)
    return _make_gather(bsz, seq, d)(table, idx)


# speedup vs baseline: 1.3719x; 1.3719x over previous
import functools

import jax
import jax.numpy as jnp
from jax import lax
from jax.experimental import pallas as pl
from jax.experimental.pallas import tpu as pltpu
from jax.experimental.pallas import tpu_sc as plsc

NC = 2
NS = 16
NW = NC * NS

DP = 128


_ROW_BLK = 8192


def _transform_body(pcat_ref, wt_ref, b_ref, out_ref):
    out_ref[...] = (
        lax.dot_general(pcat_ref[...], wt_ref[...],
                        dimension_numbers=(((0,), (0,)), ((), ())),
                        preferred_element_type=jnp.float32,
                        precision=lax.Precision.HIGHEST)
        + b_ref[...]
    )


def _transform_table(pcat, wt, b2d):
    d, v = pcat.shape
    grid = (v + _ROW_BLK - 1) // _ROW_BLK
    return pl.pallas_call(
        _transform_body,
        grid=(grid,),
        in_specs=[
            pl.BlockSpec((d, _ROW_BLK), lambda i: (0, i)),
            pl.BlockSpec((d, DP), lambda i: (0, 0)),
            pl.BlockSpec((1, DP), lambda i: (0, 0)),
        ],
        out_specs=pl.BlockSpec((_ROW_BLK, DP), lambda i: (i, 0)),
        out_shape=jax.ShapeDtypeStruct((v, DP), jnp.float32),
    )(pcat, wt, b2d)


_CHUNK = 128
_K = 2


def _make_gather(n_flat):
    per_w = n_flat // NW
    n_chunks = per_w // _CHUNK
    n_groups = n_chunks // _K
    grp = _K * _CHUNK
    mesh = plsc.VectorSubcoreMesh(
        core_axis_name="c", subcore_axis_name="s",
        num_cores=NC, num_subcores=NS)

    @functools.partial(
        pl.kernel,
        out_type=jax.ShapeDtypeStruct((n_flat, DP), jnp.float32),
        mesh=mesh,
        scratch_types=[
            pltpu.VMEM((n_chunks, _CHUNK), jnp.int32),
            pltpu.VMEM((3, grp, DP), jnp.float32),
            pltpu.SemaphoreType.DMA,
            pltpu.SemaphoreType.DMA,
            pltpu.SemaphoreType.DMA,
        ],
        compiler_params=pltpu.CompilerParams(use_tc_tiling_on_sc=True),
    )
    def gather(table_hbm, idx_hbm, out_hbm, idx_v, rows_v, *sems):
        wid = lax.axis_index("s") * NC + lax.axis_index("c")
        pltpu.sync_copy(idx_hbm.at[wid], idx_v)
        base = wid * per_w

        def fire(g, b):
            for k in range(_K):
                pltpu.async_copy(
                    table_hbm.at[idx_v.at[g * _K + k]],
                    rows_v.at[b, pl.ds(k * _CHUNK, _CHUNK)],
                    sems[b])

        def drain(b):
            for k in range(_K):
                pltpu.make_async_copy(
                    table_hbm.at[idx_v.at[0]],
                    rows_v.at[b, pl.ds(k * _CHUNK, _CHUNK)],
                    sems[b]).wait()

        def write(g, b):
            pltpu.sync_copy(rows_v.at[b],
                            out_hbm.at[pl.ds(base + g * grp, grp)])

        fire(0, 0)
        fire(1, 1)

        def body(i2, carry):
            for r in range(3):
                g = i2 * 3 + r
                fire(g + 2, (r + 2) % 3)
                drain(r)
                write(g, r)
            return carry

        n_full = (n_groups - 4) // 3
        lax.fori_loop(0, n_full, body, 0)
        for g in range(n_full * 3, n_groups):
            if g + 2 < n_groups:
                fire(g + 2, (g + 2) % 3)
            drain(g % 3)
            write(g, g % 3)

    return gather


def kernel(nodes, pca_matrix, W, b):
    bsz, seq = nodes.shape
    d = pca_matrix.shape[1]
    n_flat = bsz * seq

    wt = jnp.zeros((d, DP), jnp.float32).at[:, :d].set(W.T)
    b2d = jnp.zeros((1, DP), jnp.float32).at[:, :d].set(b)
    table = _transform_table(pca_matrix.T, wt, b2d)

    per_w = n_flat // NW
    idx = nodes.reshape(-1).astype(jnp.int32).reshape(NW, per_w // _CHUNK, _CHUNK)
    out = _make_gather(n_flat)(table, idx)
    return out[:, :d].reshape(bsz, seq, d)
